# baseline (speedup 1.0000x reference)
import functools

import jax
import jax.numpy as jnp
from jax import lax
from jax.experimental import pallas as pl
from jax.experimental.pallas import tpu as pltpu
from jax.experimental.pallas import tpu_sc as plsc

NC = 2
NS = 16
NW = NC * NS
LANES = 16
EMB = 64
BN = 32768
S = 294912
OFF = S // BN
WCOLS = 128
PW = S // NW
NWIN = PW // WCOLS
SROWS = PW // 128
BROWS = S // 128
CH = 64


def _q_body(admw_ref, itemt_ref, ehr_ref, w1_ref, q_ref, *, n_adm):
    admw = admw_ref[...].reshape(64, 1)
    ehr = ehr_ref[...]
    itemt = itemt_ref[...]
    w1 = w1_ref[...]
    v = ehr.shape[0]
    iota = lax.broadcasted_iota(jnp.int32, (admw.shape[0], v), 1)
    onehot = jnp.where(admw == iota, 1.0 / n_adm, 0.0)
    u = jnp.sum(onehot, axis=0, keepdims=True)
    y = lax.dot_general(itemt, w1, (((0,), (0,)), ((), ())),
                        preferred_element_type=jnp.float32)
    deg = jnp.sum(ehr, axis=1, keepdims=True) + 1.0
    z = jnp.dot(ehr, y, preferred_element_type=jnp.float32) + y
    node = jnp.maximum(z / jnp.maximum(deg, 1.0), 0.0)
    q_ref[...] = jnp.dot(u, node, preferred_element_type=jnp.float32)


def _dn_body(q_ref, tbl_ref, d_ref, n_ref):
    blk = tbl_ref[...]
    qv = q_ref[...]
    d = jnp.dot(qv, blk, preferred_element_type=jnp.float32)
    ones = jnp.ones((1, EMB), jnp.float32)
    n = jnp.dot(ones, blk * blk, preferred_element_type=jnp.float32)
    d_ref[...] = d.reshape(BN // 128, 128)
    n_ref[...] = n.reshape(BN // 128, 128)


def _inv_norm_scale(acc_n):
    i = plsc.bitcast(acc_n, jnp.int32)
    i = jnp.int32(0x5F3759DF) - lax.shift_right_logical(i, 1)
    y = plsc.bitcast(i, jnp.float32)
    for _ in range(3):
        y = y * (1.5 - 0.5 * acc_n * y * y)
    return jnp.minimum(1.0, y)


def _make_scdn_kernel():
    mesh = plsc.VectorSubcoreMesh(core_axis_name="c", subcore_axis_name="s")

    @functools.partial(
        pl.kernel,
        out_type=(jax.ShapeDtypeStruct((BROWS, 128), jnp.float32),
                  jax.ShapeDtypeStruct((BROWS, 128), jnp.float32)),
        mesh=mesh,
        compiler_params=pltpu.CompilerParams(needs_layout_passes=False),
        scratch_types=[
            pltpu.VMEM((EMB, 128), jnp.float32),
            pltpu.VMEM((EMB, WCOLS), jnp.float32),
            pltpu.VMEM((EMB, WCOLS), jnp.float32),
            pltpu.VMEM((SROWS, 128), jnp.float32),
            pltpu.VMEM((SROWS, 128), jnp.float32),
            pltpu.SemaphoreType.DMA,
            pltpu.SemaphoreType.DMA,
        ],
    )
    def scdn(q_hbm, tbl_hbm, db_hbm, nb_hbm,
             q_v, win0, win1, dstage, nstage, sem0, sem1):
        wid = lax.axis_index("s") * NC + lax.axis_index("c")
        pltpu.sync_copy(q_hbm, q_v)
        wins = (win0, win1)
        sems = (sem0, sem1)
        base = wid * PW

        def fire(w, b):
            c0 = pl.multiple_of(base + w * WCOLS, WCOLS)
            pltpu.async_copy(tbl_hbm.at[:, pl.ds(c0, WCOLS)], wins[b], sems[b])

        def wait(b):
            pltpu.make_async_copy(tbl_hbm.at[:, pl.ds(0, WCOLS)], wins[b],
                                  sems[b]).wait()

        z16 = jnp.zeros((LANES,), jnp.int32)

        def compute(w, b):
            win = wins[b]
            for h in range(WCOLS // 128):
                accd = [jnp.zeros((LANES,), jnp.float32) for _ in range(8)]
                accn = [jnp.zeros((LANES,), jnp.float32) for _ in range(8)]
                for j in range(EMB):
                    sp = q_v[j, pl.ds(0, LANES)]
                    for v in range(8):
                        t = win[j, pl.ds(h * 128 + v * LANES, LANES)]
                        accd[v] = accd[v] + t * sp
                        accn[v] = accn[v] + t * t
                row = w * (WCOLS // 128) + h
                for v in range(8):
                    dstage[row, pl.ds(v * LANES, LANES)] = accd[v]
                    nstage[row, pl.ds(v * LANES, LANES)] = accn[v]

        fire(0, 0)
        fire(1, 1)

        def step(i, carry):
            for b in (0, 1):
                w = i * 2 + b
                wait(b)
                compute(w, b)

                @pl.when(w + 2 < NWIN)
                def _():
                    fire(w + 2, b)
            return carry

        lax.fori_loop(0, NWIN // 2, step, 0)
        pltpu.sync_copy(dstage, db_hbm.at[pl.ds(wid * SROWS, SROWS)])
        pltpu.sync_copy(nstage, nb_hbm.at[pl.ds(wid * SROWS, SROWS)])

    return scdn


def _make_gather_kernel(b_total, arows):
    bpw = b_total // NW
    nch = bpw // CH
    mesh = plsc.VectorSubcoreMesh(core_axis_name="c", subcore_axis_name="s")

    @functools.partial(
        pl.kernel,
        out_type=jax.ShapeDtypeStruct((b_total,), jnp.float32),
        mesh=mesh,
        compiler_params=pltpu.CompilerParams(needs_layout_passes=False),
        scratch_types=[
            pltpu.VMEM((bpw // 128, 128), jnp.int32),
            pltpu.VMEM((nch, CH), jnp.int32),
            pltpu.VMEM((nch, CH), jnp.int32),
            pltpu.VMEM((CH, 128), jnp.float32),
            pltpu.VMEM((CH, 128), jnp.float32),
            pltpu.VMEM((CH, 128), jnp.float32),
            pltpu.VMEM((CH, 128), jnp.float32),
            pltpu.VMEM((CH, 128), jnp.float32),
            pltpu.VMEM((CH, 128), jnp.float32),
            pltpu.VMEM((CH, 128), jnp.float32),
            pltpu.VMEM((CH, 128), jnp.float32),
            pltpu.VMEM((bpw,), jnp.float32),
            pltpu.SemaphoreType.DMA,
            pltpu.SemaphoreType.DMA,
        ],
    )
    def gk(uid_hbm, da_hbm, na_hbm, db_hbm, nb_hbm, out_hbm,
           uid_v, tixa, tixb, ad0, ad1, an0, an1, bd0, bd1, bn0, bn1,
           out_v, sem0, sem1):
        wid = lax.axis_index("s") * NC + lax.axis_index("c")
        pltpu.sync_copy(uid_hbm.at[wid], uid_v)
        adb = (ad0, ad1)
        anb = (an0, an1)
        bdb = (bd0, bd1)
        bnb = (bn0, bn1)
        sems = (sem0, sem1)

        def load_u(k, j):
            col = k * CH + j * LANES
            return uid_v[col // 128, pl.ds(col % 128, LANES)]

        for k in range(nch):
            for j in range(CH // LANES):
                u = load_u(k, j)
                ta = jnp.maximum(
                    lax.shift_right_arithmetic(u - jnp.int32(S), 7),
                    jnp.int32(0))
                tb = jnp.minimum(lax.shift_right_logical(u, 7),
                                 jnp.int32(BROWS - 1))
                tixa[k, pl.ds(j * LANES, LANES)] = ta
                tixb[k, pl.ds(j * LANES, LANES)] = tb

        def fire(k, b):
            return (
                pltpu.async_copy(da_hbm.at[tixa.at[k]], adb[b], sems[b]),
                pltpu.async_copy(na_hbm.at[tixa.at[k]], anb[b], sems[b]),
                pltpu.async_copy(db_hbm.at[tixb.at[k]], bdb[b], sems[b]),
                pltpu.async_copy(nb_hbm.at[tixb.at[k]], bnb[b], sems[b]),
            )

        def compute(k, b):
            rid = lax.iota(jnp.int32, LANES)
            for j in range(CH // LANES):
                u = load_u(k, j)
                c = lax.bitwise_and(u, jnp.int32(127))
                use_b = u < jnp.int32(S)
                idx = [j * LANES + rid, c]
                vd = jnp.where(use_b, plsc.load_gather(bdb[b], idx),
                               plsc.load_gather(adb[b], idx))
                vn = jnp.where(use_b, plsc.load_gather(bnb[b], idx),
                               plsc.load_gather(anb[b], idx))
                out_v[pl.ds(k * CH + j * LANES, LANES)] = (
                    vd * _inv_norm_scale(vn))

        descs = [fire(0, 0), None]
        for k in range(nch):
            b = k % 2
            if k + 1 < nch:
                descs[1 - b] = fire(k + 1, 1 - b)
            for dsc in descs[b]:
                dsc.wait()
            compute(k, b)
        pltpu.sync_copy(out_v, out_hbm.at[pl.ds(wid * bpw, bpw)])

    return gk


def kernel(user_id, adm, user_table, item_table, ehr_adj, W1):
    b = user_id.shape[0]
    n_adm = adm.shape[0]
    v_users = user_table.shape[0]
    admw = jnp.pad(adm.astype(jnp.int32), (0, 64 - n_adm),
                   constant_values=-1).reshape(1, 64)
    tablet = user_table.T
    q = pl.pallas_call(
        functools.partial(_q_body, n_adm=n_adm),
        out_shape=jax.ShapeDtypeStruct((1, EMB), jnp.float32),
    )(admw, item_table.T, ehr_adj, W1)

    steps = (v_users - S + BN - 1) // BN
    arows = steps * (BN // 128)
    d2a, n2a = pl.pallas_call(
        _dn_body,
        grid=(steps,),
        in_specs=[
            pl.BlockSpec((1, EMB), lambda i: (0, 0)),
            pl.BlockSpec((EMB, BN), lambda i: (0, i + OFF)),
        ],
        out_specs=[
            pl.BlockSpec((BN // 128, 128), lambda i: (i, 0)),
            pl.BlockSpec((BN // 128, 128), lambda i: (i, 0)),
        ],
        out_shape=[
            jax.ShapeDtypeStruct((arows, 128), jnp.float32),
            jax.ShapeDtypeStruct((arows, 128), jnp.float32),
        ],
    )(q, tablet)

    qb = jnp.broadcast_to(q.reshape(EMB, 1), (EMB, 128))
    db, nb = _make_scdn_kernel()(qb, tablet)

    uid3 = user_id.astype(jnp.int32).reshape(NW, -1, 128)
    out = _make_gather_kernel(b, arows)(uid3, d2a, n2a, db, nb)
    return out.reshape(b, 1)


# speedup vs baseline: 5.2725x; 5.2725x over previous
import functools

import jax
import jax.numpy as jnp
from jax import lax
from jax.experimental import pallas as pl
from jax.experimental.pallas import tpu as pltpu
from jax.experimental.pallas import tpu_sc as plsc

NC = 2
NS = 16
NW = NC * NS
LANES = 16
EMB = 64
BN = 32768


def _dnq_body(admw_ref, itemt_ref, ehr_ref, w1_ref, tbl_ref, d_ref, n_ref,
              q_s, *, n_adm):
    @pl.when(pl.program_id(0) == 0)
    def _():
        admw = admw_ref[...].reshape(64, 1)
        ehr = ehr_ref[...]
        itemt = itemt_ref[...]
        w1 = w1_ref[...]
        v = ehr.shape[0]
        iota = lax.broadcasted_iota(jnp.int32, (admw.shape[0], v), 1)
        onehot = jnp.where(admw == iota, 1.0 / n_adm, 0.0)
        u = jnp.sum(onehot, axis=0, keepdims=True)
        y = lax.dot_general(itemt, w1, (((0,), (0,)), ((), ())),
                            preferred_element_type=jnp.float32)
        deg = jnp.sum(ehr, axis=1, keepdims=True) + 1.0
        z = jnp.dot(ehr, y, preferred_element_type=jnp.float32) + y
        node = jnp.maximum(z / jnp.maximum(deg, 1.0), 0.0)
        q_s[...] = jnp.dot(u, node, preferred_element_type=jnp.float32)

    blk = tbl_ref[...]
    qv = q_s[...]
    d = jnp.dot(qv, blk, preferred_element_type=jnp.float32)
    ones = jnp.ones((1, EMB), jnp.float32)
    n = jnp.dot(ones, blk * blk, preferred_element_type=jnp.float32)
    d_ref[...] = d.reshape(BN // 128, 128)
    n_ref[...] = n.reshape(BN // 128, 128)


def _inv_norm_scale(acc_n):
    i = plsc.bitcast(acc_n, jnp.int32)
    i = jnp.int32(0x5F3759DF) - lax.shift_right_logical(i, 1)
    y = plsc.bitcast(i, jnp.float32)
    for _ in range(3):
        y = y * (1.5 - 0.5 * acc_n * y * y)
    return jnp.minimum(1.0, y)


CH = 64


def _make_sc_kernel(b_total, drows):
    bpw = b_total // NW
    nch = bpw // CH
    mesh = plsc.VectorSubcoreMesh(core_axis_name="c", subcore_axis_name="s")

    @functools.partial(
        pl.kernel,
        out_type=jax.ShapeDtypeStruct((b_total,), jnp.float32),
        mesh=mesh,
        compiler_params=pltpu.CompilerParams(needs_layout_passes=False),
        scratch_types=[
            pltpu.VMEM((bpw // 128, 128), jnp.int32),
            pltpu.VMEM((nch, CH), jnp.int32),
            pltpu.VMEM((CH, 128), jnp.float32),
            pltpu.VMEM((CH, 128), jnp.float32),
            pltpu.VMEM((CH, 128), jnp.float32),
            pltpu.VMEM((CH, 128), jnp.float32),
            pltpu.VMEM((bpw,), jnp.float32),
            pltpu.SemaphoreType.DMA,
            pltpu.SemaphoreType.DMA,
        ],
    )
    def sc_kernel(uid_hbm, d_hbm, n_hbm, out_hbm,
                  uid_v, tix_v, dbuf0, dbuf1, nbuf0, nbuf1, out_v, sem0, sem1):
        wid = lax.axis_index("s") * NC + lax.axis_index("c")
        pltpu.sync_copy(uid_hbm.at[wid], uid_v)
        dbufs = (dbuf0, dbuf1)
        nbufs = (nbuf0, nbuf1)
        sems = (sem0, sem1)

        def load_u(k, j):
            col = k * CH + j * LANES
            return uid_v[col // 128, pl.ds(col % 128, LANES)]

        for k in range(nch):
            for j in range(CH // LANES):
                tix_v[k, pl.ds(j * LANES, LANES)] = lax.shift_right_logical(
                    load_u(k, j), 7)

        def fire(k, b):
            return (pltpu.async_copy(d_hbm.at[tix_v.at[k]], dbufs[b], sems[b]),
                    pltpu.async_copy(n_hbm.at[tix_v.at[k]], nbufs[b], sems[b]))

        def compute(k, b):
            rid = lax.iota(jnp.int32, LANES)
            for j in range(CH // LANES):
                c = lax.bitwise_and(load_u(k, j), jnp.int32(127))
                vd = plsc.load_gather(dbufs[b], [j * LANES + rid, c])
                vn = plsc.load_gather(nbufs[b], [j * LANES + rid, c])
                out_v[pl.ds(k * CH + j * LANES, LANES)] = (
                    vd * _inv_norm_scale(vn))

        descs = [fire(0, 0), None]
        for k in range(nch):
            b = k % 2
            if k + 1 < nch:
                descs[1 - b] = fire(k + 1, 1 - b)
            for dsc in descs[b]:
                dsc.wait()
            compute(k, b)
        pltpu.sync_copy(out_v, out_hbm.at[pl.ds(wid * bpw, bpw)])

    return sc_kernel


def kernel(user_id, adm, user_table, item_table, ehr_adj, W1):
    b = user_id.shape[0]
    n_adm = adm.shape[0]
    v_users = user_table.shape[0]
    admw = jnp.pad(adm.astype(jnp.int32), (0, 64 - n_adm),
                   constant_values=-1).reshape(1, 64)
    tablet = user_table.T
    steps = (v_users + BN - 1) // BN
    drows = steps * (BN // 128)
    vocab = ehr_adj.shape[0]
    d2, n2 = pl.pallas_call(
        functools.partial(_dnq_body, n_adm=n_adm),
        grid=(steps,),
        in_specs=[
            pl.BlockSpec((1, 64), lambda i: (0, 0)),
            pl.BlockSpec((EMB, vocab), lambda i: (0, 0)),
            pl.BlockSpec((vocab, vocab), lambda i: (0, 0)),
            pl.BlockSpec((EMB, EMB), lambda i: (0, 0)),
            pl.BlockSpec((EMB, BN), lambda i: (0, i)),
        ],
        out_specs=[
            pl.BlockSpec((BN // 128, 128), lambda i: (i, 0)),
            pl.BlockSpec((BN // 128, 128), lambda i: (i, 0)),
        ],
        out_shape=[
            jax.ShapeDtypeStruct((drows, 128), jnp.float32),
            jax.ShapeDtypeStruct((drows, 128), jnp.float32),
        ],
        scratch_shapes=[pltpu.VMEM((1, EMB), jnp.float32)],
    )(admw, item_table.T, ehr_adj, W1, tablet)

    uid3 = user_id.astype(jnp.int32).reshape(NW, -1, 128)
    out = _make_sc_kernel(b, drows)(uid3, d2, n2)
    return out.reshape(b, 1)


# speedup vs baseline: 5.3202x; 1.0091x over previous
import functools

import jax
import jax.numpy as jnp
from jax import lax
from jax.experimental import pallas as pl
from jax.experimental.pallas import tpu as pltpu
from jax.experimental.pallas import tpu_sc as plsc

NC = 2
NS = 16
NW = NC * NS
LANES = 16
EMB = 64
BN = 32768


def _dnq_body(admw_ref, itemt_ref, ehr_ref, w1_ref, tbl_ref, d_ref, n_ref,
              q_s, *, n_adm):
    @pl.when(pl.program_id(0) == 0)
    def _():
        admw = admw_ref[...].reshape(64, 1)
        ehr = ehr_ref[...]
        itemt = itemt_ref[...]
        w1 = w1_ref[...]
        v = ehr.shape[0]
        iota = lax.broadcasted_iota(jnp.int32, (admw.shape[0], v), 1)
        onehot = jnp.where(admw == iota, 1.0 / n_adm, 0.0)
        u = jnp.sum(onehot, axis=0, keepdims=True)
        y = lax.dot_general(itemt, w1, (((0,), (0,)), ((), ())),
                            preferred_element_type=jnp.float32)
        deg = jnp.sum(ehr, axis=1, keepdims=True) + 1.0
        z = jnp.dot(ehr, y, preferred_element_type=jnp.float32) + y
        node = jnp.maximum(z / jnp.maximum(deg, 1.0), 0.0)
        q_s[...] = jnp.dot(u, node, preferred_element_type=jnp.float32)

    blk = tbl_ref[...]
    qv = q_s[...]
    d = jnp.dot(qv, blk, preferred_element_type=jnp.float32)
    ones = jnp.ones((1, EMB), jnp.float32)
    n = jnp.dot(ones, blk * blk, preferred_element_type=jnp.float32)
    d_ref[...] = d.reshape(BN // 128, 128)
    n_ref[...] = n.reshape(BN // 128, 128)


def _inv_norm_scale(acc_n):
    i = plsc.bitcast(acc_n, jnp.int32)
    i = jnp.int32(0x5F3759DF) - lax.shift_right_logical(i, 1)
    y = plsc.bitcast(i, jnp.float32)
    for _ in range(3):
        y = y * (1.5 - 0.5 * acc_n * y * y)
    return jnp.minimum(1.0, y)


CH = 128


def _make_sc_kernel(b_total, drows):
    bpw = b_total // NW
    nch = bpw // CH
    mesh = plsc.VectorSubcoreMesh(core_axis_name="c", subcore_axis_name="s")

    @functools.partial(
        pl.kernel,
        out_type=jax.ShapeDtypeStruct((b_total,), jnp.float32),
        mesh=mesh,
        compiler_params=pltpu.CompilerParams(needs_layout_passes=False),
        scratch_types=[
            pltpu.VMEM((bpw // 128, 128), jnp.int32),
            pltpu.VMEM((nch, CH), jnp.int32),
            pltpu.VMEM((CH, 128), jnp.float32),
            pltpu.VMEM((CH, 128), jnp.float32),
            pltpu.VMEM((CH, 128), jnp.float32),
            pltpu.VMEM((CH, 128), jnp.float32),
            pltpu.VMEM((bpw,), jnp.float32),
            pltpu.SemaphoreType.DMA,
            pltpu.SemaphoreType.DMA,
        ],
    )
    def sc_kernel(uid_hbm, d_hbm, n_hbm, out_hbm,
                  uid_v, tix_v, dbuf0, dbuf1, nbuf0, nbuf1, out_v, sem0, sem1):
        wid = lax.axis_index("s") * NC + lax.axis_index("c")
        pltpu.sync_copy(uid_hbm.at[wid], uid_v)
        dbufs = (dbuf0, dbuf1)
        nbufs = (nbuf0, nbuf1)
        sems = (sem0, sem1)

        def load_u(k, j):
            col = k * CH + j * LANES
            return uid_v[col // 128, pl.ds(col % 128, LANES)]

        for k in range(nch):
            for j in range(CH // LANES):
                tix_v[k, pl.ds(j * LANES, LANES)] = lax.shift_right_logical(
                    load_u(k, j), 7)

        def fire(k, b):
            return (pltpu.async_copy(d_hbm.at[tix_v.at[k]], dbufs[b], sems[b]),
                    pltpu.async_copy(n_hbm.at[tix_v.at[k]], nbufs[b], sems[b]))

        def compute(k, b):
            rid = lax.iota(jnp.int32, LANES)
            for j in range(CH // LANES):
                c = lax.bitwise_and(load_u(k, j), jnp.int32(127))
                vd = plsc.load_gather(dbufs[b], [j * LANES + rid, c])
                vn = plsc.load_gather(nbufs[b], [j * LANES + rid, c])
                out_v[pl.ds(k * CH + j * LANES, LANES)] = (
                    vd * _inv_norm_scale(vn))

        descs = [fire(0, 0), None]
        for k in range(nch):
            b = k % 2
            if k + 1 < nch:
                descs[1 - b] = fire(k + 1, 1 - b)
            for dsc in descs[b]:
                dsc.wait()
            compute(k, b)
        pltpu.sync_copy(out_v, out_hbm.at[pl.ds(wid * bpw, bpw)])

    return sc_kernel


def kernel(user_id, adm, user_table, item_table, ehr_adj, W1):
    b = user_id.shape[0]
    n_adm = adm.shape[0]
    v_users = user_table.shape[0]
    admw = jnp.pad(adm.astype(jnp.int32), (0, 64 - n_adm),
                   constant_values=-1).reshape(1, 64)
    tablet = user_table.T
    steps = (v_users + BN - 1) // BN
    drows = steps * (BN // 128)
    vocab = ehr_adj.shape[0]
    d2, n2 = pl.pallas_call(
        functools.partial(_dnq_body, n_adm=n_adm),
        grid=(steps,),
        in_specs=[
            pl.BlockSpec((1, 64), lambda i: (0, 0)),
            pl.BlockSpec((EMB, vocab), lambda i: (0, 0)),
            pl.BlockSpec((vocab, vocab), lambda i: (0, 0)),
            pl.BlockSpec((EMB, EMB), lambda i: (0, 0)),
            pl.BlockSpec((EMB, BN), lambda i: (0, i)),
        ],
        out_specs=[
            pl.BlockSpec((BN // 128, 128), lambda i: (i, 0)),
            pl.BlockSpec((BN // 128, 128), lambda i: (i, 0)),
        ],
        out_shape=[
            jax.ShapeDtypeStruct((drows, 128), jnp.float32),
            jax.ShapeDtypeStruct((drows, 128), jnp.float32),
        ],
        scratch_shapes=[pltpu.VMEM((1, EMB), jnp.float32)],
    )(admw, item_table.T, ehr_adj, W1, tablet)

    uid3 = user_id.astype(jnp.int32).reshape(NW, -1, 128)
    out = _make_sc_kernel(b, drows)(uid3, d2, n2)
    return out.reshape(b, 1)
